# baseline (speedup 1.0000x reference)
import functools

import jax
import jax.numpy as jnp
from jax import lax
from jax.experimental import pallas as pl
from jax.experimental.pallas import tpu as pltpu
from jax.experimental.pallas import tpu_sc as plsc

N_TOK = 8192
D = 2048
BT = 512
LANES = 16
NC, NS = 2, 16
NW = NC * NS
ROWS_W = N_TOK // NW
CH = 8
NCHUNK = ROWS_W // CH
KV = D // LANES


def _lane_shuffle(s, p):
    return lax.gather(
        s, p[:, None],
        lax.GatherDimensionNumbers(
            offset_dims=(), collapsed_slice_dims=(0,), start_index_map=(0,)),
        slice_sizes=(1,),
        mode=lax.GatherScatterMode.PROMISE_IN_BOUNDS)


def _expert_tile(x_ref, w_ref, x0_ref, x1_ref):
    x = x_ref[...]
    w = w_ref[...]
    s = jnp.dot(x, w, preferred_element_type=jnp.float32)
    d = s[:, 1:2] - s[:, 0:1]
    g1 = jax.nn.sigmoid(d)
    g0 = 1.0 - g1
    pick1 = d > 0.0
    x0_ref[...] = x * jnp.where(pick1, 0.0, g0)
    x1_ref[...] = x * jnp.where(pick1, g1, 0.0)


_sc_mesh = plsc.VectorSubcoreMesh(
    core_axis_name="c", subcore_axis_name="s", num_cores=NC, num_subcores=NS
)


@functools.partial(
    pl.kernel,
    out_type=jax.ShapeDtypeStruct((N_TOK, D), jnp.float32),
    mesh=_sc_mesh,
    scratch_types=[
        pltpu.VMEM((2, CH, D), jnp.float32),
        pltpu.VMEM((2, CH, D), jnp.float32),
        pltpu.VMEM((D,), jnp.float32),
        pltpu.SemaphoreType.DMA((2,)),
        pltpu.SemaphoreType.DMA((2,)),
    ],
)
def _sc_out(x_hbm, wd_hbm, out_hbm, xv, ov, wdv, sin, sout):
    wid = lax.axis_index("s") * NC + lax.axis_index("c")
    base = wid * ROWS_W
    pltpu.sync_copy(wd_hbm, wdv)
    iot = lax.iota(jnp.int32, LANES)
    perms = [iot ^ sh for sh in (8, 4, 2, 1)]

    def cp_in(c, b):
        return pltpu.make_async_copy(
            x_hbm.at[pl.ds(base + c * CH, CH)], xv.at[b], sin.at[b])

    def cp_out(c, b):
        return pltpu.make_async_copy(
            ov.at[b], out_hbm.at[pl.ds(base + c * CH, CH)], sout.at[b])

    def compute(b):
        def dot_body(k, accs):
            o = k * LANES
            wdc = wdv[pl.ds(o, LANES)]
            return tuple(
                accs[r] + xv[b, r, pl.ds(o, LANES)] * wdc for r in range(CH))

        accs = lax.fori_loop(
            0, KV, dot_body,
            tuple(jnp.zeros((LANES,), jnp.float32) for _ in range(CH)))
        wvs = []
        for r in range(CH):
            s = accs[r]
            for p in perms:
                s = s + _lane_shuffle(s, p)
            wvs.append(1.0 / (1.0 + jnp.exp(-jnp.abs(s))))

        def scale_body(k, _):
            o = k * LANES
            for r in range(CH):
                ov[b, r, pl.ds(o, LANES)] = xv[b, r, pl.ds(o, LANES)] * wvs[r]
            return 0

        lax.fori_loop(0, KV, scale_body, 0)

    cp_in(0, 0).start()
    cp_in(1, 1).start()

    def outer(g, _):
        for b in range(2):
            c = 2 * g + b
            cp_in(c, b).wait()

            @pl.when(g > 0)
            def _wait_prev_out():
                cp_out(c, b).wait()

            compute(b)
            cp_out(c, b).start()

            @pl.when(c + 2 < NCHUNK)
            def _start_next_in():
                cp_in(c + 2, b).start()
        return 0

    lax.fori_loop(0, NCHUNK // 2, outer, 0)
    cp_out(0, 0).wait()
    cp_out(0, 1).wait()


def kernel(x, W):
    wd = W[:, 1] - W[:, 0]
    out = _sc_out(x, wd)
    grid = (N_TOK // BT,)
    shp = jax.ShapeDtypeStruct((N_TOK, D), x.dtype)
    x0, x1 = pl.pallas_call(
        _expert_tile,
        grid=grid,
        in_specs=[
            pl.BlockSpec((BT, D), lambda i: (i, 0)),
            pl.BlockSpec((D, 2), lambda i: (0, 0)),
        ],
        out_specs=[
            pl.BlockSpec((BT, D), lambda i: (i, 0)),
            pl.BlockSpec((BT, D), lambda i: (i, 0)),
        ],
        out_shape=[shp, shp],
    )(x, W)
    return (x0, x1, out)


# speedup vs baseline: 1.4364x; 1.4364x over previous
import jax
import jax.numpy as jnp
from jax.experimental import pallas as pl

N_TOK = 8192
D = 2048
BT = 256


def _router_tile(x_ref, w_ref, x0_ref, x1_ref, out_ref):
    x = x_ref[...]
    w = w_ref[...]
    s = jnp.dot(x, w, preferred_element_type=jnp.float32)
    d = s[:, 1:2] - s[:, 0:1]
    g1 = jax.nn.sigmoid(d)
    g0 = 1.0 - g1
    pick1 = d > 0.0
    w0 = jnp.where(pick1, 0.0, g0)
    w1 = jnp.where(pick1, g1, 0.0)
    x0_ref[...] = x * w0
    x1_ref[...] = x * w1
    out_ref[...] = x * (w0 + w1)


def kernel(x, W):
    grid = (N_TOK // BT,)
    shp = jax.ShapeDtypeStruct((N_TOK, D), x.dtype)
    x0, x1, out = pl.pallas_call(
        _router_tile,
        grid=grid,
        in_specs=[
            pl.BlockSpec((BT, D), lambda i: (i, 0)),
            pl.BlockSpec((D, 2), lambda i: (0, 0)),
        ],
        out_specs=[
            pl.BlockSpec((BT, D), lambda i: (i, 0)),
            pl.BlockSpec((BT, D), lambda i: (i, 0)),
            pl.BlockSpec((BT, D), lambda i: (i, 0)),
        ],
        out_shape=[shp, shp, shp],
    )(x, W)
    return (x0, x1, out)


# speedup vs baseline: 1.4972x; 1.0423x over previous
import jax
import jax.numpy as jnp
from jax.experimental import pallas as pl

N_TOK = 8192
D = 2048
BT = 512


def _router_tile(x_ref, w_ref, x0_ref, x1_ref, out_ref):
    x = x_ref[...]
    w = w_ref[...]
    s = jnp.dot(x, w, preferred_element_type=jnp.float32)
    d = s[:, 1:2] - s[:, 0:1]
    g1 = jax.nn.sigmoid(d)
    g0 = 1.0 - g1
    pick1 = d > 0.0
    w0 = jnp.where(pick1, 0.0, g0)
    w1 = jnp.where(pick1, g1, 0.0)
    x0_ref[...] = x * w0
    x1_ref[...] = x * w1
    out_ref[...] = x * (w0 + w1)


def kernel(x, W):
    grid = (N_TOK // BT,)
    shp = jax.ShapeDtypeStruct((N_TOK, D), x.dtype)
    x0, x1, out = pl.pallas_call(
        _router_tile,
        grid=grid,
        in_specs=[
            pl.BlockSpec((BT, D), lambda i: (i, 0)),
            pl.BlockSpec((D, 2), lambda i: (0, 0)),
        ],
        out_specs=[
            pl.BlockSpec((BT, D), lambda i: (i, 0)),
            pl.BlockSpec((BT, D), lambda i: (i, 0)),
            pl.BlockSpec((BT, D), lambda i: (i, 0)),
        ],
        out_shape=[shp, shp, shp],
    )(x, W)
    return (x0, x1, out)
